# baseline (speedup 1.0000x reference)
import functools

import jax
import jax.numpy as jnp
from jax.experimental import pallas as pl
from jax.experimental.pallas import tpu as pltpu


def _pick_block(n, prefer):
    for b in prefer:
        if n % b == 0:
            return b
    return n


def _body(emb_ref, w1_ref, b1_ref, lnw_ref, lnb_ref, w2_ref, b2_ref, v_ref,
          out_ref, h_ref, tn_ref, mu_ref, isd_ref, l_ref,
          *, nk, bk, nv, inv_temp):
    j = pl.program_id(0)

    @pl.when(j < nk)
    def _phase1():
        part = jnp.dot(
            emb_ref[:, pl.ds(j * bk, bk)], w1_ref[...],
            preferred_element_type=jnp.float32,
        )

        @pl.when(j == 0)
        def _():
            h_ref[...] = part

        @pl.when(j > 0)
        def _():
            h_ref[...] = h_ref[...] + part

    @pl.when(j == nk)
    def _ln_stats():
        h = h_ref[...] + b1_ref[...]
        mu = jnp.mean(h, axis=-1, keepdims=True)
        var = jnp.mean((h - mu) * (h - mu), axis=-1, keepdims=True)
        mu_ref[...] = mu
        isd_ref[...] = jax.lax.rsqrt(var + 1e-5)

    @pl.when(jnp.logical_and(j >= nk, j < 2 * nk))
    def _phase2():
        k = j - nk
        hs = (
            h_ref[:, pl.ds(k * bk, bk)]
            + b1_ref[:, pl.ds(k * bk, bk)]
            - mu_ref[...]
        ) * isd_ref[...] * lnw_ref[:, pl.ds(k * bk, bk)] + lnb_ref[:, pl.ds(k * bk, bk)]
        gs = 0.5 * hs * (1.0 + jax.lax.erf(hs * 0.7071067811865476))
        part = jnp.dot(gs, w2_ref[...], preferred_element_type=jnp.float32)

        @pl.when(k == 0)
        def _():
            tn_ref[...] = part

        @pl.when(k > 0)
        def _():
            tn_ref[...] = tn_ref[...] + part

        @pl.when(k == nk - 1)
        def _():
            t = emb_ref[...] + 0.2 * (tn_ref[...] + b2_ref[...])
            nrm = jnp.sqrt(jnp.sum(t * t, axis=-1, keepdims=True))
            tn_ref[...] = t / jnp.maximum(nrm, 1e-12)
            l_ref[...] = jnp.zeros_like(l_ref)
            out_ref[...] = jnp.zeros_like(out_ref)

    @pl.when(j >= 2 * nk)
    def _phase3():
        vb = v_ref[...]
        ss = jnp.sum(vb * vb, axis=-1, keepdims=True)
        rn = jax.lax.rsqrt(jnp.maximum(ss, 1e-24))
        sims = jax.lax.dot_general(
            tn_ref[...], vb, (((1,), (1,)), ((), ())),
            preferred_element_type=jnp.float32,
        )
        s = sims * jnp.transpose(rn)
        p = jnp.exp((s - 1.0) * inv_temp)
        l_ref[...] = l_ref[...] + jnp.sum(p, axis=-1, keepdims=True)
        out_ref[...] = out_ref[...] + jnp.dot(
            p, vb, preferred_element_type=jnp.float32
        )

        @pl.when(j == 2 * nk + nv - 1)
        def _():
            out_ref[...] = out_ref[...] / l_ref[...]


def kernel(embeddings, token_ids, W1, b1, ln_w, ln_b, W2, b2, vocab_embeds):
    del token_ids
    n, d = embeddings.shape
    h_dim = W1.shape[1]
    v = vocab_embeds.shape[0]

    bk = _pick_block(d, (256, 128, 64))
    bv = _pick_block(v, (640, 512, 500, 256, 128, 64))
    nk = d // bk
    nv = v // bv

    f32 = jnp.float32
    b1r = b1.reshape(1, h_dim)
    lnwr = ln_w.reshape(1, h_dim)
    lnbr = ln_b.reshape(1, h_dim)
    b2r = b2.reshape(1, d)

    out = pl.pallas_call(
        functools.partial(_body, nk=nk, bk=bk, nv=nv, inv_temp=10.0),
        grid=(2 * nk + nv,),
        in_specs=[
            pl.BlockSpec((n, d), lambda j: (0, 0)),
            pl.BlockSpec((bk, h_dim), lambda j: (jnp.minimum(j, nk - 1), 0)),
            pl.BlockSpec((1, h_dim), lambda j: (0, 0)),
            pl.BlockSpec((1, h_dim), lambda j: (0, 0)),
            pl.BlockSpec((1, h_dim), lambda j: (0, 0)),
            pl.BlockSpec(
                (bk, d),
                lambda j: (jnp.minimum(jnp.maximum(j - nk, 0), nk - 1), 0),
            ),
            pl.BlockSpec((1, d), lambda j: (0, 0)),
            pl.BlockSpec((bv, d), lambda j: (jnp.maximum(j - 2 * nk, 0), 0)),
        ],
        out_specs=pl.BlockSpec((n, d), lambda j: (0, 0)),
        out_shape=jax.ShapeDtypeStruct((n, d), f32),
        scratch_shapes=[
            pltpu.VMEM((n, h_dim), f32),
            pltpu.VMEM((n, d), f32),
            pltpu.VMEM((n, 1), f32),
            pltpu.VMEM((n, 1), f32),
            pltpu.VMEM((n, 1), f32),
        ],
    )(embeddings, W1, b1r, lnwr, lnbr, W2, b2r, vocab_embeds)
    return out


# speedup vs baseline: 1.1038x; 1.1038x over previous
import functools

import jax
import jax.numpy as jnp
from jax.experimental import pallas as pl
from jax.experimental.pallas import tpu as pltpu


def _pick_block(n, prefer):
    for b in prefer:
        if n % b == 0:
            return b
    return n


def _mlp_body(emb_ref, w1_ref, b1_ref, lnw_ref, lnb_ref, w2_ref, b2_ref,
              t_ref, h_ref, mu_ref, isd_ref, *, nk, bk):
    j = pl.program_id(0)

    @pl.when(j < nk)
    def _phase1():
        part = jnp.dot(
            emb_ref[:, pl.ds(j * bk, bk)], w1_ref[...],
            preferred_element_type=jnp.float32,
        )

        @pl.when(j == 0)
        def _():
            h_ref[...] = part

        @pl.when(j > 0)
        def _():
            h_ref[...] = h_ref[...] + part

    @pl.when(j == nk)
    def _ln_stats():
        h = h_ref[...] + b1_ref[...]
        mu = jnp.mean(h, axis=-1, keepdims=True)
        var = jnp.mean((h - mu) * (h - mu), axis=-1, keepdims=True)
        mu_ref[...] = mu
        isd_ref[...] = jax.lax.rsqrt(var + 1e-5)

    @pl.when(j >= nk)
    def _phase2():
        k = j - nk
        hs = (
            h_ref[:, pl.ds(k * bk, bk)]
            + b1_ref[:, pl.ds(k * bk, bk)]
            - mu_ref[...]
        ) * isd_ref[...] * lnw_ref[:, pl.ds(k * bk, bk)] + lnb_ref[:, pl.ds(k * bk, bk)]
        gs = 0.5 * hs * (1.0 + jax.lax.erf(hs * 0.7071067811865476))
        part = jnp.dot(
            gs, w2_ref[...],
            preferred_element_type=jnp.float32,
        )

        @pl.when(j == nk)
        def _():
            t_ref[...] = part

        @pl.when(j > nk)
        def _():
            t_ref[...] = t_ref[...] + part

        @pl.when(j == 2 * nk - 1)
        def _():
            t = emb_ref[...] + 0.2 * (t_ref[...] + b2_ref[...])
            nrm = jnp.sqrt(jnp.sum(t * t, axis=-1, keepdims=True))
            t_ref[...] = t / jnp.maximum(nrm, 1e-12)


def _flash_body(tn_ref, v_ref, out_ref, l_ref, *, nsteps, inv_temp):
    i = pl.program_id(0)

    @pl.when(i == 0)
    def _init():
        l_ref[...] = jnp.zeros_like(l_ref)
        out_ref[...] = jnp.zeros_like(out_ref)

    vb = v_ref[...]
    ss = jnp.sum(vb * vb, axis=-1, keepdims=True)
    rn = jax.lax.rsqrt(jnp.maximum(ss, 1e-24))
    sims = jax.lax.dot_general(
        tn_ref[...], vb, (((1,), (1,)), ((), ())),
        preferred_element_type=jnp.float32,
    )
    s = sims * jnp.transpose(rn)
    p = jnp.exp((s - 1.0) * inv_temp)
    l_ref[...] = l_ref[...] + jnp.sum(p, axis=-1, keepdims=True)
    out_ref[...] = out_ref[...] + jnp.dot(
        p, vb, preferred_element_type=jnp.float32
    )

    @pl.when(i == nsteps - 1)
    def _fin():
        out_ref[...] = out_ref[...] / l_ref[...]


def kernel(embeddings, token_ids, W1, b1, ln_w, ln_b, W2, b2, vocab_embeds):
    del token_ids
    n, d = embeddings.shape
    h_dim = W1.shape[1]
    v = vocab_embeds.shape[0]

    bh = _pick_block(h_dim, (512, 256, 128, 64))
    bd = _pick_block(d, (512, 256, 128, 64))
    bv = _pick_block(v, (1000, 800, 512, 500, 256, 128, 64))

    f32 = jnp.float32
    b1r = b1.reshape(1, h_dim)
    lnwr = ln_w.reshape(1, h_dim)
    lnbr = ln_b.reshape(1, h_dim)
    b2r = b2.reshape(1, d)

    bk = _pick_block(d, (512, 256, 128, 64))
    nk = d // bk
    t = pl.pallas_call(
        functools.partial(_mlp_body, nk=nk, bk=bk),
        grid=(2 * nk,),
        in_specs=[
            pl.BlockSpec((n, d), lambda j: (0, 0)),
            pl.BlockSpec((bk, h_dim), lambda j: (jnp.minimum(j, nk - 1), 0)),
            pl.BlockSpec((1, h_dim), lambda j: (0, 0)),
            pl.BlockSpec((1, h_dim), lambda j: (0, 0)),
            pl.BlockSpec((1, h_dim), lambda j: (0, 0)),
            pl.BlockSpec((bk, d), lambda j: (jnp.maximum(j - nk, 0), 0)),
            pl.BlockSpec((1, d), lambda j: (0, 0)),
        ],
        out_specs=pl.BlockSpec((n, d), lambda j: (0, 0)),
        out_shape=jax.ShapeDtypeStruct((n, d), f32),
        scratch_shapes=[
            pltpu.VMEM((n, h_dim), f32),
            pltpu.VMEM((n, 1), f32),
            pltpu.VMEM((n, 1), f32),
        ],
    )(embeddings, W1, b1r, lnwr, lnbr, W2, b2r)

    nsteps = v // bv
    out = pl.pallas_call(
        functools.partial(_flash_body, nsteps=nsteps, inv_temp=10.0),
        grid=(nsteps,),
        in_specs=[
            pl.BlockSpec((n, d), lambda i: (0, 0)),
            pl.BlockSpec((bv, d), lambda i: (i, 0)),
        ],
        out_specs=pl.BlockSpec((n, d), lambda i: (0, 0)),
        out_shape=jax.ShapeDtypeStruct((n, d), f32),
        scratch_shapes=[
            pltpu.VMEM((n, 1), f32),
        ],
    )(t, vocab_embeds)
    return out
